# baseline (speedup 1.0000x reference)
import functools

import jax
import jax.numpy as jnp
from jax import lax
from jax.experimental import pallas as pl
from jax.experimental.pallas import tpu as pltpu
from jax.experimental.pallas import tpu_sc as plsc

_SC_CORES = 2
_SC_SUBCORES = 16


def _proj_body(seq_ref, wl_ref, wr_ref, ws_ref, pos_ref, ca_ref,
               l_ref, r_ref, st_ref, tab_ref):
    s = seq_ref[...]
    dn = (((1,), (0,)), ((), ()))
    l_ref[...] = lax.dot_general(s, wl_ref[...], dn,
                                 preferred_element_type=jnp.float32)
    r_ref[...] = lax.dot_general(s, wr_ref[...], dn,
                                 preferred_element_type=jnp.float32)
    st_ref[...] = lax.dot_general(s, ws_ref[...], dn,
                                  preferred_element_type=jnp.float32)
    nbin, nca = pos_ref.shape[0], ca_ref.shape[0]
    d = pos_ref.shape[1]
    pad = jnp.zeros((d - nbin - nca, d), jnp.float32)
    tab_ref[...] = jnp.concatenate(
        [pos_ref[...], ca_ref[...], pad], axis=0).astype(jnp.bfloat16)


def _pair_body(blk, nbin_hi, idx_row_ref, idx_col_ref, ca_ref, l_ref, r_ref,
               tab_ref, out_ref):
    L = l_ref.shape[0]
    D = l_ref.shape[1]
    ii = idx_col_ref[...]
    jj = idx_row_ref[...]
    p = jnp.clip(jj - ii + 32, 0, nbin_hi)
    q = (p << 7) | (ca_ref[0] + (nbin_hi + 1))
    q3 = q[:, :, None]
    col = lax.broadcasted_iota(jnp.int32, (blk, L, D), 2)
    oh = (col == (q3 >> 7)) | (col == (q3 & 127))
    ohb = oh.astype(jnp.bfloat16).reshape(blk * L, D)
    g = lax.dot_general(ohb, tab_ref[...], (((1,), (0,)), ((), ())),
                        preferred_element_type=jnp.float32)
    out_ref[...] = (g.reshape(blk, L, D)
                    + l_ref[...][None, :, :]
                    + r_ref[...][:, None, :])


def _bcast_body(nb, qs_ref, out_ref):
    out_ref[...] = jnp.broadcast_to(qs_ref[...][None], (nb,) + qs_ref.shape)


def kernel(msa, seq, idx, CA_dist_matrix, emb_q_w, emb_left_w, emb_right_w,
           emb_state_w, pos_emb_w, ca_emb_w):
    B, N = msa.shape[0], msa.shape[1]
    L, DI = seq.shape[1], seq.shape[2]
    DM, DP, DS = emb_q_w.shape[1], emb_left_w.shape[1], emb_state_w.shape[1]
    NBIN = pos_emb_w.shape[0]
    NCA = ca_emb_w.shape[0]

    seq2 = seq.reshape(B * L, DI)
    idx2 = idx.reshape(B * L).astype(jnp.int32)
    ca3 = CA_dist_matrix.astype(jnp.int32)

    f32 = jnp.float32
    left, right, state, tab = pl.pallas_call(
        _proj_body,
        out_shape=(
            jax.ShapeDtypeStruct((B * L, DP), f32),
            jax.ShapeDtypeStruct((B * L, DP), f32),
            jax.ShapeDtypeStruct((B * L, DS), f32),
            jax.ShapeDtypeStruct((DP, DP), jnp.bfloat16),
        ),
    )(seq2, emb_left_w, emb_right_w, emb_state_w,
      pos_emb_w, ca_emb_w)

    BLK = 32
    grid = (L // BLK,)
    pair = pl.pallas_call(
        functools.partial(_pair_body, BLK, NBIN - 1),
        grid=grid,
        in_specs=[
            pl.BlockSpec((1, L), lambda i: (0, 0)),
            pl.BlockSpec((BLK, 1), lambda i: (i, 0)),
            pl.BlockSpec((1, BLK, L), lambda i: (0, i, 0)),
            pl.BlockSpec((L, DP), lambda i: (0, 0)),
            pl.BlockSpec((BLK, DP), lambda i: (i, 0)),
            pl.BlockSpec((DP, DP), lambda i: (0, 0)),
        ],
        out_specs=pl.BlockSpec((BLK, L, DP), lambda i: (i, 0, 0)),
        out_shape=jax.ShapeDtypeStruct((L, L, DP), f32),
    )(idx2.reshape(1, L), idx2.reshape(L, 1), ca3, left, right, tab)

    nw = _SC_CORES * _SC_SUBCORES
    lw = 24
    mh = DM // 2
    mesh = plsc.VectorSubcoreMesh(
        core_axis_name="c", subcore_axis_name="s",
        num_cores=_SC_CORES, num_subcores=_SC_SUBCORES)

    K = 8
    G = N // K
    nmc = mh // 16

    def _sc_msa(seq_hbm, wq_hbm, out_hbm, seq_v, w_v, buf, sem):
        wid = lax.axis_index("s") * _SC_CORES + lax.axis_index("c")
        l0 = (wid // 2) * lw
        m0 = (wid % 2) * mh
        pltpu.sync_copy(seq_hbm.at[pl.ds(l0, lw)], seq_v)
        pltpu.sync_copy(wq_hbm.at[:, pl.ds(m0, mh)], w_v)

        def lbody(l, _):
            v0 = seq_v[l, pl.ds(0, 16)]
            v1 = seq_v[l, pl.ds(16, 16)]
            s = [v0[k] if k < 16 else v1[k - 16] for k in range(DI)]
            for mc in range(nmc):
                acc = s[0] * w_v[0, pl.ds(mc * 16, 16)]
                for k in range(1, DI):
                    acc = acc + s[k] * w_v[k, pl.ds(mc * 16, 16)]
                buf[l, pl.ds(mc * 16, 16)] = acc
            return ()

        lax.fori_loop(0, lw, lbody, ())

        def fire(gbase):
            for b in range(K):
                pltpu.async_copy(
                    buf,
                    out_hbm.at[gbase + b, pl.ds(l0, lw), pl.ds(m0, mh)],
                    sem)

        def drain_one():
            pltpu.make_async_copy(
                buf, out_hbm.at[0, pl.ds(l0, lw), pl.ds(m0, mh)],
                sem).wait()

        fire(0)

        def body(g, _):
            fire((g + 1) * K)
            for _b in range(K):
                drain_one()
            return ()

        lax.fori_loop(0, G - 1, body, ())
        for _b in range(K):
            drain_one()

    msa_out = pl.kernel(
        _sc_msa,
        out_type=jax.ShapeDtypeStruct((N, B * L, DM), f32),
        mesh=mesh,
        scratch_types=[
            pltpu.VMEM((lw, 32), f32),
            pltpu.VMEM((DI, mh), f32),
            pltpu.VMEM((lw, mh), f32),
            pltpu.SemaphoreType.DMA,
        ],
    )(jnp.pad(seq2, ((0, 0), (0, 32 - DI))), emb_q_w)

    return (msa_out.reshape(B, N, L, DM),
            pair.reshape(B, L, L, DP),
            state.reshape(B, L, DS))


# speedup vs baseline: 1.0877x; 1.0877x over previous
import functools

import jax
import jax.numpy as jnp
from jax import lax
from jax.experimental import pallas as pl
from jax.experimental.pallas import tpu as pltpu
from jax.experimental.pallas import tpu_sc as plsc

_SC_CORES = 2
_SC_SUBCORES = 16


def _proj_body(seq_ref, wq_ref, wl_ref, wr_ref, ws_ref, pos_ref, ca_ref,
               qs_ref, l_ref, r_ref, st_ref, tab_ref):
    s = seq_ref[...]
    dn = (((1,), (0,)), ((), ()))
    qs_ref[...] = lax.dot_general(s, wq_ref[...], dn,
                                  preferred_element_type=jnp.float32)
    l_ref[...] = lax.dot_general(s, wl_ref[...], dn,
                                 preferred_element_type=jnp.float32)
    r_ref[...] = lax.dot_general(s, wr_ref[...], dn,
                                 preferred_element_type=jnp.float32)
    st_ref[...] = lax.dot_general(s, ws_ref[...], dn,
                                  preferred_element_type=jnp.float32)
    nbin, nca = pos_ref.shape[0], ca_ref.shape[0]
    d = pos_ref.shape[1]
    pad = jnp.zeros((d - nbin - nca, d), jnp.float32)
    tab_ref[...] = jnp.concatenate(
        [pos_ref[...], ca_ref[...], pad], axis=0).astype(jnp.bfloat16)


def _pair_body(blk, nbin_hi, idx_row_ref, idx_col_ref, ca_ref, l_ref, r_ref,
               tab_ref, out_ref):
    L = l_ref.shape[0]
    D = l_ref.shape[1]
    ii = idx_col_ref[...]
    jj = idx_row_ref[...]
    p = jnp.clip(jj - ii + 32, 0, nbin_hi)
    ca = lax.bitcast_convert_type(ca_ref[0], jnp.int32)
    q = (p << 7) | (ca + (nbin_hi + 1))
    q3 = q[:, :, None]
    col = lax.broadcasted_iota(jnp.int32, (blk, L, D), 2)
    oh = (col == (q3 >> 7)) | (col == (q3 & 127))
    ohb = oh.astype(jnp.bfloat16).reshape(blk * L, D)
    g = lax.dot_general(ohb, tab_ref[...], (((1,), (0,)), ((), ())),
                        preferred_element_type=jnp.float32)
    out_ref[...] = (g.reshape(blk, L, D)
                    + l_ref[...][None, :, :]
                    + r_ref[...][:, None, :])


def _bcast_body(nb, qs_ref, out_ref):
    out_ref[...] = jnp.broadcast_to(qs_ref[...][None], (nb,) + qs_ref.shape)


def kernel(msa, seq, idx, CA_dist_matrix, emb_q_w, emb_left_w, emb_right_w,
           emb_state_w, pos_emb_w, ca_emb_w):
    B, N = msa.shape[0], msa.shape[1]
    L, DI = seq.shape[1], seq.shape[2]
    DM, DP, DS = emb_q_w.shape[1], emb_left_w.shape[1], emb_state_w.shape[1]
    NBIN = pos_emb_w.shape[0]
    NCA = ca_emb_w.shape[0]

    seq2 = seq.reshape(B * L, DI)
    idx2 = idx.reshape(B * L).astype(jnp.int32)
    ca3 = lax.bitcast_convert_type(CA_dist_matrix.astype(jnp.int32),
                                   jnp.float32)

    f32 = jnp.float32
    qs, left, right, state, tab = pl.pallas_call(
        _proj_body,
        out_shape=(
            jax.ShapeDtypeStruct((B * L, DM), f32),
            jax.ShapeDtypeStruct((B * L, DP), f32),
            jax.ShapeDtypeStruct((B * L, DP), f32),
            jax.ShapeDtypeStruct((B * L, DS), f32),
            jax.ShapeDtypeStruct((DP, DP), jnp.bfloat16),
        ),
    )(seq2, emb_q_w, emb_left_w, emb_right_w, emb_state_w,
      pos_emb_w, ca_emb_w)

    BLK = 32
    grid = (L // BLK,)
    pair = pl.pallas_call(
        functools.partial(_pair_body, BLK, NBIN - 1),
        grid=grid,
        in_specs=[
            pl.BlockSpec((1, L), lambda i: (0, 0)),
            pl.BlockSpec((BLK, 1), lambda i: (i, 0)),
            pl.BlockSpec((1, BLK, L), lambda i: (0, i, 0)),
            pl.BlockSpec((L, DP), lambda i: (0, 0)),
            pl.BlockSpec((BLK, DP), lambda i: (i, 0)),
            pl.BlockSpec((DP, DP), lambda i: (0, 0)),
        ],
        out_specs=pl.BlockSpec((BLK, L, DP), lambda i: (i, 0, 0)),
        out_shape=jax.ShapeDtypeStruct((L, L, DP), f32),
    )(idx2.reshape(1, L), idx2.reshape(L, 1), ca3, left, right, tab)

    nw = _SC_CORES * _SC_SUBCORES
    lw = 24
    n_lc = (B * L) // lw
    NR = N // (nw // n_lc)
    mesh = plsc.VectorSubcoreMesh(
        core_axis_name="c", subcore_axis_name="s",
        num_cores=_SC_CORES, num_subcores=_SC_SUBCORES)

    K = 8
    G = NR // K

    def _sc_bcast(qs_hbm, out_hbm, buf, sem):
        wid = lax.axis_index("s") * _SC_CORES + lax.axis_index("c")
        l0 = (wid // 2) * lw
        n_base = (wid % 2) * NR
        pltpu.sync_copy(qs_hbm.at[pl.ds(l0, lw)], buf)

        def fire(gbase):
            for b in range(K):
                pltpu.async_copy(
                    buf, out_hbm.at[n_base + gbase + b, pl.ds(l0, lw)], sem)

        def drain_one():
            pltpu.make_async_copy(buf, out_hbm.at[0, pl.ds(l0, lw)],
                                  sem).wait()

        fire(0)

        def body(g, _):
            fire((g + 1) * K)
            for _b in range(K):
                drain_one()
            return ()

        lax.fori_loop(0, G - 1, body, ())
        for _b in range(K):
            drain_one()

    msa_out = pl.kernel(
        _sc_bcast,
        out_type=jax.ShapeDtypeStruct((N, B * L, DM), f32),
        mesh=mesh,
        scratch_types=[
            pltpu.VMEM((lw, DM), f32),
            pltpu.SemaphoreType.DMA,
        ],
    )(qs)

    return (msa_out.reshape(B, N, L, DM),
            pair.reshape(B, L, L, DP),
            state.reshape(B, L, DS))


# speedup vs baseline: 1.0885x; 1.0007x over previous
import functools

import jax
import jax.numpy as jnp
from jax import lax
from jax.experimental import pallas as pl
from jax.experimental.pallas import tpu as pltpu
from jax.experimental.pallas import tpu_sc as plsc

_SC_CORES = 2
_SC_SUBCORES = 16


def _proj_body(seq_ref, wq_ref, wl_ref, wr_ref, ws_ref, pos_ref, ca_ref,
               qs_ref, l_ref, r_ref, st_ref, tab_ref):
    s = seq_ref[...]
    dn = (((1,), (0,)), ((), ()))
    qs_ref[...] = lax.dot_general(s, wq_ref[...], dn,
                                  preferred_element_type=jnp.float32)
    l_ref[...] = lax.dot_general(s, wl_ref[...], dn,
                                 preferred_element_type=jnp.float32)
    r_ref[...] = lax.dot_general(s, wr_ref[...], dn,
                                 preferred_element_type=jnp.float32)
    st_ref[...] = lax.dot_general(s, ws_ref[...], dn,
                                  preferred_element_type=jnp.float32)
    nbin, nca = pos_ref.shape[0], ca_ref.shape[0]
    d = pos_ref.shape[1]
    pad = jnp.zeros((d - nbin - nca, d), jnp.float32)
    tab_ref[...] = jnp.concatenate(
        [pos_ref[...], ca_ref[...], pad], axis=0).astype(jnp.bfloat16)


def _pair_body(blk, nbin_hi, idx_row_ref, idx_col_ref, ca_ref, l_ref, r_ref,
               tab_ref, out_ref):
    L = l_ref.shape[0]
    D = l_ref.shape[1]
    ii = idx_col_ref[...]
    jj = idx_row_ref[...]
    p = jnp.clip(jj - ii + 32, 0, nbin_hi)
    q = (p << 7) | (ca_ref[0] + (nbin_hi + 1))
    q3 = q[:, :, None]
    col = lax.broadcasted_iota(jnp.int32, (blk, L, D), 2)
    oh = (col == (q3 >> 7)) | (col == (q3 & 127))
    ohb = oh.astype(jnp.bfloat16).reshape(blk * L, D)
    g = lax.dot_general(ohb, tab_ref[...], (((1,), (0,)), ((), ())),
                        preferred_element_type=jnp.float32)
    out_ref[...] = (g.reshape(blk, L, D)
                    + l_ref[...][None, :, :]
                    + r_ref[...][:, None, :])


def _bcast_body(nb, qs_ref, out_ref):
    out_ref[...] = jnp.broadcast_to(qs_ref[...][None], (nb,) + qs_ref.shape)


def kernel(msa, seq, idx, CA_dist_matrix, emb_q_w, emb_left_w, emb_right_w,
           emb_state_w, pos_emb_w, ca_emb_w):
    B, N = msa.shape[0], msa.shape[1]
    L, DI = seq.shape[1], seq.shape[2]
    DM, DP, DS = emb_q_w.shape[1], emb_left_w.shape[1], emb_state_w.shape[1]
    NBIN = pos_emb_w.shape[0]
    NCA = ca_emb_w.shape[0]

    seq2 = seq.reshape(B * L, DI)
    idx2 = idx.reshape(B * L).astype(jnp.int32)
    ca3 = CA_dist_matrix.astype(jnp.int32)

    f32 = jnp.float32
    qs, left, right, state, tab = pl.pallas_call(
        _proj_body,
        out_shape=(
            jax.ShapeDtypeStruct((B * L, DM), f32),
            jax.ShapeDtypeStruct((B * L, DP), f32),
            jax.ShapeDtypeStruct((B * L, DP), f32),
            jax.ShapeDtypeStruct((B * L, DS), f32),
            jax.ShapeDtypeStruct((DP, DP), jnp.bfloat16),
        ),
    )(seq2, emb_q_w, emb_left_w, emb_right_w, emb_state_w,
      pos_emb_w, ca_emb_w)

    BLK = 32
    grid = (L // BLK,)
    pair = pl.pallas_call(
        functools.partial(_pair_body, BLK, NBIN - 1),
        grid=grid,
        in_specs=[
            pl.BlockSpec((1, L), lambda i: (0, 0)),
            pl.BlockSpec((BLK, 1), lambda i: (i, 0)),
            pl.BlockSpec((1, BLK, L), lambda i: (0, i, 0)),
            pl.BlockSpec((L, DP), lambda i: (0, 0)),
            pl.BlockSpec((BLK, DP), lambda i: (i, 0)),
            pl.BlockSpec((DP, DP), lambda i: (0, 0)),
        ],
        out_specs=pl.BlockSpec((BLK, L, DP), lambda i: (i, 0, 0)),
        out_shape=jax.ShapeDtypeStruct((L, L, DP), f32),
    )(idx2.reshape(1, L), idx2.reshape(L, 1), ca3, left, right, tab)

    nw = _SC_CORES * _SC_SUBCORES
    lw = 24
    n_lc = (B * L) // lw
    NR = N // (nw // n_lc)
    mesh = plsc.VectorSubcoreMesh(
        core_axis_name="c", subcore_axis_name="s",
        num_cores=_SC_CORES, num_subcores=_SC_SUBCORES)

    K = 8
    G = NR // K

    def _sc_bcast(qs_hbm, out_hbm, buf, sem):
        wid = lax.axis_index("s") * _SC_CORES + lax.axis_index("c")
        l0 = (wid // 2) * lw
        n_base = (wid % 2) * NR
        pltpu.sync_copy(qs_hbm.at[pl.ds(l0, lw)], buf)

        def fire(gbase):
            for b in range(K):
                pltpu.async_copy(
                    buf, out_hbm.at[n_base + gbase + b, pl.ds(l0, lw)], sem)

        def drain_one():
            pltpu.make_async_copy(buf, out_hbm.at[0, pl.ds(l0, lw)],
                                  sem).wait()

        fire(0)

        def body(g, _):
            fire((g + 1) * K)
            for _b in range(K):
                drain_one()
            return ()

        lax.fori_loop(0, G - 1, body, ())
        for _b in range(K):
            drain_one()

    msa_out = pl.kernel(
        _sc_bcast,
        out_type=jax.ShapeDtypeStruct((N, B * L, DM), f32),
        mesh=mesh,
        scratch_types=[
            pltpu.VMEM((lw, DM), f32),
            pltpu.SemaphoreType.DMA,
        ],
    )(qs)

    return (msa_out.reshape(B, N, L, DM),
            pair.reshape(B, L, L, DP),
            state.reshape(B, L, DS))


# speedup vs baseline: 1.0991x; 1.0098x over previous
import functools

import jax
import jax.numpy as jnp
from jax import lax
from jax.experimental import pallas as pl
from jax.experimental.pallas import tpu as pltpu
from jax.experimental.pallas import tpu_sc as plsc

_SC_CORES = 2
_SC_SUBCORES = 16


def _proj_body(seq_ref, wq_ref, wl_ref, wr_ref, ws_ref, pos_ref, ca_ref,
               qs_ref, l_ref, r_ref, st_ref, tab_ref):
    s = seq_ref[...]
    dn = (((1,), (0,)), ((), ()))
    qs_ref[...] = lax.dot_general(s, wq_ref[...], dn,
                                  preferred_element_type=jnp.float32)
    l_ref[...] = lax.dot_general(s, wl_ref[...], dn,
                                 preferred_element_type=jnp.float32)
    r_ref[...] = lax.dot_general(s, wr_ref[...], dn,
                                 preferred_element_type=jnp.float32)
    st_ref[...] = lax.dot_general(s, ws_ref[...], dn,
                                  preferred_element_type=jnp.float32)
    nbin, nca = pos_ref.shape[0], ca_ref.shape[0]
    d = pos_ref.shape[1]
    pad = jnp.zeros((d - nbin - nca, d), jnp.float32)
    tab_ref[...] = jnp.concatenate(
        [pos_ref[...], ca_ref[...], pad], axis=0).astype(jnp.bfloat16)


def _pair_body(blk, nbin_hi, idx_row_ref, idx_col_ref, ca_ref, l_ref, r_ref,
               tab_ref, out_ref):
    L = l_ref.shape[0]
    D = l_ref.shape[1]
    ii = idx_col_ref[...]
    jj = idx_row_ref[...]
    p = jnp.clip(jj - ii + 32, 0, nbin_hi)
    q = (p << 7) | (ca_ref[0] + (nbin_hi + 1))
    q3 = q[:, :, None]
    col = lax.broadcasted_iota(jnp.int32, (blk, L, D), 2)
    oh = (col == (q3 >> 7)) | (col == (q3 & 127))
    ohb = oh.astype(jnp.bfloat16).reshape(blk * L, D)
    g = lax.dot_general(ohb, tab_ref[...], (((1,), (0,)), ((), ())),
                        preferred_element_type=jnp.float32)
    out_ref[...] = (g.reshape(blk, L, D)
                    + l_ref[...][None, :, :]
                    + r_ref[...][:, None, :])


def _bcast_body(nb, qs_ref, out_ref):
    out_ref[...] = jnp.broadcast_to(qs_ref[...][None], (nb,) + qs_ref.shape)


def kernel(msa, seq, idx, CA_dist_matrix, emb_q_w, emb_left_w, emb_right_w,
           emb_state_w, pos_emb_w, ca_emb_w):
    B, N = msa.shape[0], msa.shape[1]
    L, DI = seq.shape[1], seq.shape[2]
    DM, DP, DS = emb_q_w.shape[1], emb_left_w.shape[1], emb_state_w.shape[1]
    NBIN = pos_emb_w.shape[0]
    NCA = ca_emb_w.shape[0]

    seq2 = seq.reshape(B * L, DI)
    idx2 = idx.reshape(B * L).astype(jnp.int32)
    ca3 = CA_dist_matrix.astype(jnp.int32)

    f32 = jnp.float32
    qs, left, right, state, tab = pl.pallas_call(
        _proj_body,
        out_shape=(
            jax.ShapeDtypeStruct((B * L, DM), f32),
            jax.ShapeDtypeStruct((B * L, DP), f32),
            jax.ShapeDtypeStruct((B * L, DP), f32),
            jax.ShapeDtypeStruct((B * L, DS), f32),
            jax.ShapeDtypeStruct((DP, DP), jnp.bfloat16),
        ),
    )(seq2, emb_q_w, emb_left_w, emb_right_w, emb_state_w,
      pos_emb_w, ca_emb_w)

    BLK = 64
    grid = (L // BLK,)
    pair = pl.pallas_call(
        functools.partial(_pair_body, BLK, NBIN - 1),
        grid=grid,
        in_specs=[
            pl.BlockSpec((1, L), lambda i: (0, 0)),
            pl.BlockSpec((BLK, 1), lambda i: (i, 0)),
            pl.BlockSpec((1, BLK, L), lambda i: (0, i, 0)),
            pl.BlockSpec((L, DP), lambda i: (0, 0)),
            pl.BlockSpec((BLK, DP), lambda i: (i, 0)),
            pl.BlockSpec((DP, DP), lambda i: (0, 0)),
        ],
        out_specs=pl.BlockSpec((BLK, L, DP), lambda i: (i, 0, 0)),
        out_shape=jax.ShapeDtypeStruct((L, L, DP), f32),
    )(idx2.reshape(1, L), idx2.reshape(L, 1), ca3, left, right, tab)

    nw = _SC_CORES * _SC_SUBCORES
    lw = 24
    n_lc = (B * L) // lw
    NR = N // (nw // n_lc)
    mesh = plsc.VectorSubcoreMesh(
        core_axis_name="c", subcore_axis_name="s",
        num_cores=_SC_CORES, num_subcores=_SC_SUBCORES)

    K = 8
    G = NR // K

    def _sc_bcast(qs_hbm, out_hbm, buf, sem):
        wid = lax.axis_index("s") * _SC_CORES + lax.axis_index("c")
        l0 = (wid // 2) * lw
        n_base = (wid % 2) * NR
        pltpu.sync_copy(qs_hbm.at[pl.ds(l0, lw)], buf)

        def fire(gbase):
            for b in range(K):
                pltpu.async_copy(
                    buf, out_hbm.at[n_base + gbase + b, pl.ds(l0, lw)], sem)

        def drain_one():
            pltpu.make_async_copy(buf, out_hbm.at[0, pl.ds(l0, lw)],
                                  sem).wait()

        fire(0)

        def body(g, _):
            fire((g + 1) * K)
            for _b in range(K):
                drain_one()
            return ()

        lax.fori_loop(0, G - 1, body, ())
        for _b in range(K):
            drain_one()

    msa_out = pl.kernel(
        _sc_bcast,
        out_type=jax.ShapeDtypeStruct((N, B * L, DM), f32),
        mesh=mesh,
        scratch_types=[
            pltpu.VMEM((lw, DM), f32),
            pltpu.SemaphoreType.DMA,
        ],
    )(qs)

    return (msa_out.reshape(B, N, L, DM),
            pair.reshape(B, L, L, DP),
            state.reshape(B, L, DS))
